# baseline (speedup 1.0000x reference)
import functools

import jax
import jax.numpy as jnp
from jax import lax
from jax.experimental import pallas as pl
from jax.experimental.pallas import tpu as pltpu
from jax.experimental.pallas import tpu_sc as plsc

D = 768
BATCH = 4
SEQ = 2048
B_TOTAL = BATCH * SEQ
L = 16
NC = 2
NS = 16
NW = NC * NS
B_PER_W = B_TOTAL // NW
W_PER_ROW = SEQ // B_PER_W
C = 16
NCHUNK = B_PER_W // C
NBUF = 4

_mesh = plsc.VectorSubcoreMesh(core_axis_name="c", subcore_axis_name="s")


@functools.partial(
    pl.kernel,
    mesh=_mesh,
    out_type=jax.ShapeDtypeStruct((BATCH, SEQ, D), jnp.float32),
    scratch_types=(
        [pltpu.VMEM((B_PER_W,), jnp.int32)] * 2
        + [pltpu.VMEM((C, D), jnp.float32)] * (2 * NBUF)
        + [pltpu.SemaphoreType.DMA] * (3 * NBUF)
    ),
)
def _embed_lookup(ids_hbm, pids_hbm, wtab_hbm, ptab_hbm, out_hbm,
                  widx, pidx, *bufs_and_sems):
    wbuf = bufs_and_sems[0:NBUF]
    pbuf = bufs_and_sems[NBUF:2 * NBUF]
    wsem = bufs_and_sems[2 * NBUF:3 * NBUF]
    psem = bufs_and_sems[3 * NBUF:4 * NBUF]
    osem = bufs_and_sems[4 * NBUF:5 * NBUF]

    wid = lax.axis_index("s") * NC + lax.axis_index("c")
    row = wid // W_PER_ROW
    col = (wid % W_PER_ROW) * B_PER_W
    pltpu.sync_copy(ids_hbm.at[row, pl.ds(col, B_PER_W)], widx)
    pltpu.sync_copy(pids_hbm.at[row, pl.ds(col, B_PER_W)], pidx)

    def issue_gathers(g):
        b = g % NBUF
        wd = pltpu.async_copy(
            wtab_hbm.at[widx.at[pl.ds(g * C, C)]], wbuf[b], wsem[b])
        pd = pltpu.async_copy(
            ptab_hbm.at[pidx.at[pl.ds(g * C, C)]], pbuf[b], psem[b])
        return wd, pd

    wd = [None] * NCHUNK
    pd = [None] * NCHUNK
    od = [None] * NCHUNK
    for k in range(NBUF - 1):
        wd[k], pd[k] = issue_gathers(k)

    for g in range(NCHUNK):
        b = g % NBUF
        gn = g + NBUF - 1
        if gn < NCHUNK:
            if g >= 1:
                od[g - 1].wait()
            wd[gn], pd[gn] = issue_gathers(gn)
        wd[g].wait()
        pd[g].wait()

        def add_row(r, carry, _wb=wbuf[b], _pb=pbuf[b]):
            for j in range(D // L):
                s = _wb[r, pl.ds(j * L, L)] + _pb[r, pl.ds(j * L, L)]
                _wb[r, pl.ds(j * L, L)] = s
            return carry

        lax.fori_loop(0, C, add_row, 0)
        od[g] = pltpu.async_copy(
            wbuf[b], out_hbm.at[row, pl.ds(col + g * C, C)], osem[b])

    for g in range(NCHUNK - NBUF, NCHUNK):
        od[g].wait()


def kernel(input_ids, position_ids, attention_mask, word_embeddings,
           position_embeddings):
    return _embed_lookup(input_ids, position_ids, word_embeddings,
                         position_embeddings)


# speedup vs baseline: 1.0425x; 1.0425x over previous
import functools

import jax
import jax.numpy as jnp
from jax import lax
from jax.experimental import pallas as pl
from jax.experimental.pallas import tpu as pltpu
from jax.experimental.pallas import tpu_sc as plsc

D = 768
BATCH = 4
SEQ = 2048
B_TOTAL = BATCH * SEQ
L = 16
NC = 2
NS = 16
NW = NC * NS
B_PER_W = B_TOTAL // NW
W_PER_ROW = SEQ // B_PER_W
C = 32
NCHUNK = B_PER_W // C
NBUF = 2

_mesh = plsc.VectorSubcoreMesh(core_axis_name="c", subcore_axis_name="s")


@functools.partial(
    pl.kernel,
    mesh=_mesh,
    out_type=jax.ShapeDtypeStruct((BATCH, SEQ, D), jnp.float32),
    scratch_types=(
        [pltpu.VMEM((B_PER_W,), jnp.int32)] * 2
        + [pltpu.VMEM((C, D), jnp.float32)] * (2 * NBUF)
        + [pltpu.SemaphoreType.DMA] * (3 * NBUF)
    ),
)
def _embed_lookup(ids_hbm, pids_hbm, wtab_hbm, ptab_hbm, out_hbm,
                  widx, pidx, *bufs_and_sems):
    wbuf = bufs_and_sems[0:NBUF]
    pbuf = bufs_and_sems[NBUF:2 * NBUF]
    wsem = bufs_and_sems[2 * NBUF:3 * NBUF]
    psem = bufs_and_sems[3 * NBUF:4 * NBUF]
    osem = bufs_and_sems[4 * NBUF:5 * NBUF]

    wid = lax.axis_index("s") * NC + lax.axis_index("c")
    row = wid // W_PER_ROW
    col = (wid % W_PER_ROW) * B_PER_W
    pltpu.sync_copy(ids_hbm.at[row, pl.ds(col, B_PER_W)], widx)
    pltpu.sync_copy(pids_hbm.at[row, pl.ds(col, B_PER_W)], pidx)

    def issue_word_gather(g):
        b = g % NBUF
        return pltpu.async_copy(
            wtab_hbm.at[widx.at[pl.ds(g * C, C)]], wbuf[b], wsem[b])

    def issue_pos_gather(g):
        b = g % NBUF
        return pltpu.async_copy(
            ptab_hbm.at[pidx.at[pl.ds(g * C, C)]], pbuf[b], psem[b])

    wd = [None] * NCHUNK
    pd = [None] * NCHUNK
    od = [None] * NCHUNK
    wd[0] = issue_word_gather(0)
    pd[0] = issue_pos_gather(0)

    for g in range(NCHUNK):
        b = g % NBUF
        if g + 1 < NCHUNK:
            wd[g + 1] = issue_word_gather(g + 1)
            if g >= 1:
                od[g - 1].wait()
            pd[g + 1] = issue_pos_gather(g + 1)
        wd[g].wait()
        pd[g].wait()

        def add_row(r, carry, _wb=wbuf[b], _pb=pbuf[b]):
            for j in range(D // L):
                s = _wb[r, pl.ds(j * L, L)] + _pb[r, pl.ds(j * L, L)]
                _pb[r, pl.ds(j * L, L)] = s
            return carry

        lax.fori_loop(0, C, add_row, 0)
        od[g] = pltpu.async_copy(
            pbuf[b], out_hbm.at[row, pl.ds(col + g * C, C)], osem[b])

    od[NCHUNK - 2].wait()
    od[NCHUNK - 1].wait()


def kernel(input_ids, position_ids, attention_mask, word_embeddings,
           position_embeddings):
    return _embed_lookup(input_ids, position_ids, word_embeddings,
                         position_embeddings)
